# baseline (speedup 1.0000x reference)
import functools

import jax
import jax.numpy as jnp
from jax import lax
from jax.experimental import pallas as pl
from jax.experimental.pallas import tpu as pltpu
from jax.experimental.pallas import tpu_sc as plsc

N_NODES = 10000
D = 256
DQ = 64
BN = 1000
N_TILES = 16
E_TOT = 160000 + N_NODES
EDGE_BATCH = 112
N_BATCH = 96
CHUNK = N_BATCH * EDGE_BATCH
E_PAD = N_TILES * CHUNK
NPAD = 10112
ZROWS = NPAD // N_TILES
EPI = 32
BN2 = 632


def _mm1_body(x_ref, w_ref, att2_ref, q0, q1, q2, q3, a2_ref):
    h = jnp.dot(x_ref[...], w_ref[...], preferred_element_type=jnp.float32)
    q0[...] = h[:, 0 * DQ:1 * DQ]
    q1[...] = h[:, 1 * DQ:2 * DQ]
    q2[...] = h[:, 2 * DQ:3 * DQ]
    q3[...] = h[:, 3 * DQ:4 * DQ]
    a2_ref[...] = jnp.dot(h, att2_ref[...], preferred_element_type=jnp.float32)


def _mm1(x, W, att2):
    return pl.pallas_call(
        _mm1_body,
        grid=(N_NODES // BN,),
        in_specs=[
            pl.BlockSpec((BN, D), lambda i: (i, 0)),
            pl.BlockSpec((D, D), lambda i: (0, 0)),
            pl.BlockSpec((D, 8), lambda i: (0, 0)),
        ],
        out_specs=[pl.BlockSpec((BN, DQ), lambda i: (i, 0))] * 4
        + [pl.BlockSpec((BN, 8), lambda i: (i, 0))],
        out_shape=[jax.ShapeDtypeStruct((NPAD, DQ), jnp.float32)] * 4
        + [jax.ShapeDtypeStruct((N_NODES, 8), jnp.float32)],
    )(x, W, att2)


def _mm2_body(x0, x1, x2, x3, w_ref, att2_ref, q0, q1, q2, q3, a2_ref):
    h = jnp.dot(jnp.maximum(x0[...], 0.0), w_ref[0 * DQ:1 * DQ, :],
                preferred_element_type=jnp.float32)
    h += jnp.dot(jnp.maximum(x1[...], 0.0), w_ref[1 * DQ:2 * DQ, :],
                 preferred_element_type=jnp.float32)
    h += jnp.dot(jnp.maximum(x2[...], 0.0), w_ref[2 * DQ:3 * DQ, :],
                 preferred_element_type=jnp.float32)
    h += jnp.dot(jnp.maximum(x3[...], 0.0), w_ref[3 * DQ:4 * DQ, :],
                 preferred_element_type=jnp.float32)
    q0[...] = h[:, 0 * DQ:1 * DQ]
    q1[...] = h[:, 1 * DQ:2 * DQ]
    q2[...] = h[:, 2 * DQ:3 * DQ]
    q3[...] = h[:, 3 * DQ:4 * DQ]
    a2_ref[...] = jnp.dot(h, att2_ref[...], preferred_element_type=jnp.float32)


def _mm2(xq, W, att2):
    return pl.pallas_call(
        _mm2_body,
        grid=(NPAD // BN2,),
        in_specs=[pl.BlockSpec((BN2, DQ), lambda i: (i, 0))] * 4
        + [
            pl.BlockSpec((D, D), lambda i: (0, 0)),
            pl.BlockSpec((D, 8), lambda i: (0, 0)),
        ],
        out_specs=[pl.BlockSpec((BN2, DQ), lambda i: (i, 0))] * 4
        + [pl.BlockSpec((BN2, 8), lambda i: (i, 0))],
        out_shape=[jax.ShapeDtypeStruct((NPAD, DQ), jnp.float32)] * 4
        + [jax.ShapeDtypeStruct((NPAD, 8), jnp.float32)],
    )(*xq, W, att2)


DW = DQ + 16


def _edge_body(hq0, hq1, hq2, hq3, asrc_h, adst_h, srcb_h, dstb_h,
               gmax_h, bias_h, oq0, oq1, oq2, oq3,
               src_v, dst_v, wb0, wb1, asrc_v, adst_v, gbuf0, gbuf1,
               sbuf0, sbuf1, gmax_v, bias_v, obuf, acc_sh,
               gsem0, gsem1, ssem0, ssem1):
    c = lax.axis_index("c")
    s = lax.axis_index("s")
    gbuf = (gbuf0, gbuf1)
    sbuf = (sbuf0, sbuf1)
    gsem = (gsem0, gsem1)
    wb = (wb0, wb1)

    pltpu.sync_copy(srcb_h.at[s], src_v)
    pltpu.sync_copy(dstb_h.at[s], dst_v)
    pltpu.sync_copy(asrc_h, asrc_v)
    pltpu.sync_copy(adst_h, adst_v)
    pltpu.sync_copy(gmax_h, gmax_v)

    zero16 = jnp.zeros((16,), jnp.float32)
    lanes = lax.iota(jnp.int32, 16)
    zbase = s * ZROWS
    gv = gmax_v[...]
    ntail = ZROWS - (ZROWS // EPI) * EPI

    def _zrow(r, _):
        for k in range(DW // 16):
            obuf[r, pl.ds(k * 16, 16)] = zero16
        return 0

    for q in range(2):

        def _issue_gather(bidx, X):
            @pl.when(c == 0)
            def _():
                pltpu.async_copy(
                    (hq0 if q == 0 else hq1).at[src_v.at[bidx]],
                    gbuf[X], gsem[X])

            @pl.when(c == 1)
            def _():
                pltpu.async_copy(
                    (hq2 if q == 0 else hq3).at[src_v.at[bidx]],
                    gbuf[X], gsem[X])

        _issue_gather(0, 0)
        _issue_gather(1, 1)

        @pl.when(c == 0)
        def _():
            pltpu.sync_copy(bias_h.at[pl.ds(q * DQ, DQ)], bias_v)

        @pl.when(c == 1)
        def _():
            pltpu.sync_copy(bias_h.at[pl.ds((2 + q) * DQ, DQ)], bias_v)

        lax.fori_loop(0, EPI, _zrow, 0)

        def _zcopy(t, _):
            pltpu.sync_copy(obuf, acc_sh.at[pl.ds(zbase + t * EPI, EPI)])
            return 0

        lax.fori_loop(0, ZROWS // EPI, _zcopy, 0)
        pltpu.sync_copy(obuf.at[pl.ds(0, ntail)],
                        acc_sh.at[pl.ds(zbase + ZROWS - ntail, ntail)])
        plsc.subcore_barrier()

        def _batch(g, _):
            for X in range(2):
                b = 2 * g + X
                pltpu.make_async_copy(
                    hq0.at[src_v.at[0]], gbuf[X], gsem[X]).wait()

                eid0 = (s * CHUNK + b * EDGE_BATCH)
                for k in range(EDGE_BATCH // 16):
                    sv = src_v[b, pl.ds(k * 16, 16)]
                    dv = dst_v[b, pl.ds(k * 16, 16)]
                    al = (plsc.load_gather(asrc_v, [sv])
                          + plsc.load_gather(adst_v, [dv]))
                    al = jnp.where(al >= 0.0, al, al * 0.2)
                    wv = jnp.exp(al - gv)
                    eid = jnp.full((16,), eid0 + k * 16, jnp.int32) + lanes
                    wv = jnp.where(eid < E_TOT, wv, 0.0)
                    wb[X][pl.ds(k * 16, 16)] = wv

                def _srow(r2, _):
                    for rr in range(2):
                        r = 2 * r2 + rr
                        wv = plsc.load_gather(
                            wb[X], [jnp.full((16,), r, jnp.int32)])
                        sbuf[X][r, pl.ds(DQ, 16)] = wv
                        for k in range(DQ // 16):
                            sbuf[X][r, pl.ds(k * 16, 16)] = (
                                gbuf[X][r, pl.ds(k * 16, 16)] * wv)
                    return 0

                lax.fori_loop(0, EDGE_BATCH // 2, _srow, 0)

                pltpu.sync_copy(sbuf[X], acc_sh.at[dst_v.at[b]], add=True)

                @pl.when(g < N_BATCH // 2 - 1)
                def _():
                    _issue_gather(b + 2, X)
            return 0

        lax.fori_loop(0, N_BATCH // 2, _batch, 0)
        plsc.subcore_barrier()

        def _echunk(ro, sz):
            pltpu.sync_copy(acc_sh.at[pl.ds(ro, sz)], obuf.at[pl.ds(0, sz)])

            def _erow(r, _):
                rv = 1.0 / (obuf[r, pl.ds(DQ, 16)] + 1e-16)
                for k in range(DQ // 16):
                    gbuf0[r, pl.ds(k * 16, 16)] = (
                        obuf[r, pl.ds(k * 16, 16)] * rv
                        + bias_v[pl.ds(k * 16, 16)])
                return 0

            lax.fori_loop(0, sz, _erow, 0)

            @pl.when(c == 0)
            def _():
                if q == 0:
                    pltpu.sync_copy(gbuf0.at[pl.ds(0, sz)], oq0.at[pl.ds(ro, sz)])
                else:
                    pltpu.sync_copy(gbuf0.at[pl.ds(0, sz)], oq1.at[pl.ds(ro, sz)])

            @pl.when(c == 1)
            def _():
                if q == 0:
                    pltpu.sync_copy(gbuf0.at[pl.ds(0, sz)], oq2.at[pl.ds(ro, sz)])
                else:
                    pltpu.sync_copy(gbuf0.at[pl.ds(0, sz)], oq3.at[pl.ds(ro, sz)])

        def _et(t, _):
            _echunk(zbase + t * EPI, EPI)
            return 0

        lax.fori_loop(0, ZROWS // EPI, _et, 0)
        _echunk(zbase + ZROWS - ntail, ntail)

        if q == 0:
            plsc.subcore_barrier()


def _edge_sc(hq, asrc, adst, srcb, dstb, gmax16, bias):
    mesh = plsc.VectorSubcoreMesh(core_axis_name="c", subcore_axis_name="s")
    f32 = jnp.float32
    kern = functools.partial(
        pl.kernel,
        mesh=mesh,
        compiler_params=pltpu.CompilerParams(
            needs_layout_passes=False, use_tc_tiling_on_sc=False),
        out_type=[jax.ShapeDtypeStruct((NPAD, DQ), f32)] * 4,
        scratch_types=[
            pltpu.VMEM((N_BATCH, EDGE_BATCH), jnp.int32),
            pltpu.VMEM((N_BATCH, EDGE_BATCH), jnp.int32),
            pltpu.VMEM((EDGE_BATCH,), f32),
            pltpu.VMEM((EDGE_BATCH,), f32),
            pltpu.VMEM((N_NODES,), f32),
            pltpu.VMEM((N_NODES,), f32),
            pltpu.VMEM((EDGE_BATCH, DQ), f32),
            pltpu.VMEM((EDGE_BATCH, DQ), f32),
            pltpu.VMEM((EDGE_BATCH, DW), f32),
            pltpu.VMEM((EDGE_BATCH, DW), f32),
            pltpu.VMEM((16,), f32),
            pltpu.VMEM((DQ,), f32),
            pltpu.VMEM((EPI, DW), f32),
            pltpu.VMEM_SHARED((NPAD, DW), f32),
            pltpu.SemaphoreType.DMA,
            pltpu.SemaphoreType.DMA,
            pltpu.SemaphoreType.DMA,
            pltpu.SemaphoreType.DMA,
        ],
    )(_edge_body)
    return kern(*hq, asrc, adst, srcb, dstb, gmax16, bias)


def _gmax16(asrc, adst):
    m = jnp.max(asrc) + jnp.max(adst)
    m = jnp.where(m >= 0.0, m, m * 0.2)
    return jnp.full((16,), m, jnp.float32)


def kernel(x, edge_index, W1, att_src1, att_dst1, b1, W2, att_src2, att_dst2, b2):
    loop = jnp.arange(N_NODES, dtype=edge_index.dtype)
    padi = jnp.zeros((E_PAD - E_TOT,), edge_index.dtype)
    srcb = jnp.concatenate([edge_index[0], loop, padi]).reshape(
        N_TILES, N_BATCH, EDGE_BATCH)
    dstb = jnp.concatenate([edge_index[1], loop, padi]).reshape(
        N_TILES, N_BATCH, EDGE_BATCH)

    att2_1 = jnp.zeros((D, 8), jnp.float32).at[:, 0].set(att_src1).at[:, 1].set(att_dst1)
    att2_2 = jnp.zeros((D, 8), jnp.float32).at[:, 0].set(att_src2).at[:, 1].set(att_dst2)

    *hq, a2 = _mm1(x, W1, att2_1)
    asrc, adst = a2[:, 0], a2[:, 1]
    o1 = _edge_sc(hq, asrc, adst, srcb, dstb, _gmax16(asrc, adst), b1)

    *h2q, a2b = _mm2(o1, W2, att2_2)
    asrc2, adst2 = a2b[:N_NODES, 0], a2b[:N_NODES, 1]
    o2 = _edge_sc(h2q, asrc2, adst2, srcb, dstb, _gmax16(asrc2, adst2), b2)
    return jnp.concatenate([q[:N_NODES] for q in o2], axis=1)


# speedup vs baseline: 1.0889x; 1.0889x over previous
import functools

import jax
import jax.numpy as jnp
from jax import lax
from jax.experimental import pallas as pl
from jax.experimental.pallas import tpu as pltpu
from jax.experimental.pallas import tpu_sc as plsc

N_NODES = 10000
D = 256
DQ = 64
BN = 1000
N_TILES = 16
E_TOT = 160000 + N_NODES
EDGE_BATCH = 112
N_BATCH = 96
CHUNK = N_BATCH * EDGE_BATCH
E_PAD = N_TILES * CHUNK
NPAD = 10112
ZROWS = NPAD // N_TILES
EPI = 32
BN2 = 632


def _mm1_body(x_ref, w_ref, att2_ref, q0, q1, q2, q3, a2_ref):
    h = jnp.dot(x_ref[...], w_ref[...], preferred_element_type=jnp.float32)
    q0[...] = h[:, 0 * DQ:1 * DQ]
    q1[...] = h[:, 1 * DQ:2 * DQ]
    q2[...] = h[:, 2 * DQ:3 * DQ]
    q3[...] = h[:, 3 * DQ:4 * DQ]
    a2_ref[...] = jnp.dot(h, att2_ref[...], preferred_element_type=jnp.float32)


def _mm1(x, W, att2):
    return pl.pallas_call(
        _mm1_body,
        grid=(N_NODES // BN,),
        in_specs=[
            pl.BlockSpec((BN, D), lambda i: (i, 0)),
            pl.BlockSpec((D, D), lambda i: (0, 0)),
            pl.BlockSpec((D, 8), lambda i: (0, 0)),
        ],
        out_specs=[pl.BlockSpec((BN, DQ), lambda i: (i, 0))] * 4
        + [pl.BlockSpec((BN, 8), lambda i: (i, 0))],
        out_shape=[jax.ShapeDtypeStruct((NPAD, DQ), jnp.float32)] * 4
        + [jax.ShapeDtypeStruct((N_NODES, 8), jnp.float32)],
    )(x, W, att2)


def _mm2_body(x0, x1, x2, x3, w_ref, att2_ref, q0, q1, q2, q3, a2_ref):
    h = jnp.dot(jnp.maximum(x0[...], 0.0), w_ref[0 * DQ:1 * DQ, :],
                preferred_element_type=jnp.float32)
    h += jnp.dot(jnp.maximum(x1[...], 0.0), w_ref[1 * DQ:2 * DQ, :],
                 preferred_element_type=jnp.float32)
    h += jnp.dot(jnp.maximum(x2[...], 0.0), w_ref[2 * DQ:3 * DQ, :],
                 preferred_element_type=jnp.float32)
    h += jnp.dot(jnp.maximum(x3[...], 0.0), w_ref[3 * DQ:4 * DQ, :],
                 preferred_element_type=jnp.float32)
    q0[...] = h[:, 0 * DQ:1 * DQ]
    q1[...] = h[:, 1 * DQ:2 * DQ]
    q2[...] = h[:, 2 * DQ:3 * DQ]
    q3[...] = h[:, 3 * DQ:4 * DQ]
    a2_ref[...] = jnp.dot(h, att2_ref[...], preferred_element_type=jnp.float32)


def _mm2(xq, W, att2):
    return pl.pallas_call(
        _mm2_body,
        grid=(NPAD // BN2,),
        in_specs=[pl.BlockSpec((BN2, DQ), lambda i: (i, 0))] * 4
        + [
            pl.BlockSpec((D, D), lambda i: (0, 0)),
            pl.BlockSpec((D, 8), lambda i: (0, 0)),
        ],
        out_specs=[pl.BlockSpec((BN2, DQ), lambda i: (i, 0))] * 4
        + [pl.BlockSpec((BN2, 8), lambda i: (i, 0))],
        out_shape=[jax.ShapeDtypeStruct((NPAD, DQ), jnp.float32)] * 4
        + [jax.ShapeDtypeStruct((NPAD, 8), jnp.float32)],
    )(*xq, W, att2)


DW = DQ + 16


def _edge_body(hq0, hq1, hq2, hq3, asrc_h, adst_h, srcb_h, dstb_h,
               gmax_h, bias_h, oq0, oq1, oq2, oq3,
               src_v, dst_v, wb0, wb1, asrc_v, adst_v, gbuf0, gbuf1,
               sbuf0, sbuf1, gmax_v, bias_v, obuf, acc_sh,
               gsem0, gsem1, ssem0, ssem1):
    c = lax.axis_index("c")
    s = lax.axis_index("s")
    gbuf = (gbuf0, gbuf1)
    sbuf = (sbuf0, sbuf1)
    gsem = (gsem0, gsem1)
    wb = (wb0, wb1)
    ssem = (ssem0, ssem1)

    pltpu.sync_copy(srcb_h.at[s], src_v)
    pltpu.sync_copy(dstb_h.at[s], dst_v)
    pltpu.sync_copy(asrc_h, asrc_v)
    pltpu.sync_copy(adst_h, adst_v)
    pltpu.sync_copy(gmax_h, gmax_v)

    zero16 = jnp.zeros((16,), jnp.float32)
    lanes = lax.iota(jnp.int32, 16)
    zbase = s * ZROWS
    gv = gmax_v[...]
    ntail = ZROWS - (ZROWS // EPI) * EPI

    def _zrow(r, _):
        for k in range(DW // 16):
            obuf[r, pl.ds(k * 16, 16)] = zero16
        return 0

    for q in range(2):

        def _issue_gather(bidx, X):
            @pl.when(c == 0)
            def _():
                pltpu.async_copy(
                    (hq0 if q == 0 else hq1).at[src_v.at[bidx]],
                    gbuf[X], gsem[X])

            @pl.when(c == 1)
            def _():
                pltpu.async_copy(
                    (hq2 if q == 0 else hq3).at[src_v.at[bidx]],
                    gbuf[X], gsem[X])

        _issue_gather(0, 0)
        _issue_gather(1, 1)

        @pl.when(c == 0)
        def _():
            pltpu.sync_copy(bias_h.at[pl.ds(q * DQ, DQ)], bias_v)

        @pl.when(c == 1)
        def _():
            pltpu.sync_copy(bias_h.at[pl.ds((2 + q) * DQ, DQ)], bias_v)

        lax.fori_loop(0, EPI, _zrow, 0)

        def _zcopy(t, _):
            pltpu.sync_copy(obuf, acc_sh.at[pl.ds(zbase + t * EPI, EPI)])
            return 0

        lax.fori_loop(0, ZROWS // EPI, _zcopy, 0)
        pltpu.sync_copy(obuf.at[pl.ds(0, ntail)],
                        acc_sh.at[pl.ds(zbase + ZROWS - ntail, ntail)])
        plsc.subcore_barrier()

        def _batch(g, _):
            for X in range(2):
                b = 2 * g + X
                pltpu.make_async_copy(
                    hq0.at[src_v.at[0]], gbuf[X], gsem[X]).wait()
                @pl.when(g > 0)
                def _():
                    pltpu.make_async_copy(
                        sbuf[X], acc_sh.at[dst_v.at[0]], ssem[X]).wait()

                eid0 = (s * CHUNK + b * EDGE_BATCH)
                for k in range(EDGE_BATCH // 16):
                    sv = src_v[b, pl.ds(k * 16, 16)]
                    dv = dst_v[b, pl.ds(k * 16, 16)]
                    al = (plsc.load_gather(asrc_v, [sv])
                          + plsc.load_gather(adst_v, [dv]))
                    al = jnp.where(al >= 0.0, al, al * 0.2)
                    wv = jnp.exp(al - gv)
                    eid = jnp.full((16,), eid0 + k * 16, jnp.int32) + lanes
                    wv = jnp.where(eid < E_TOT, wv, 0.0)
                    wb[X][pl.ds(k * 16, 16)] = wv

                def _srow(r2, _):
                    for rr in range(2):
                        r = 2 * r2 + rr
                        wv = plsc.load_gather(
                            wb[X], [jnp.full((16,), r, jnp.int32)])
                        sbuf[X][r, pl.ds(DQ, 16)] = wv
                        for k in range(DQ // 16):
                            sbuf[X][r, pl.ds(k * 16, 16)] = (
                                gbuf[X][r, pl.ds(k * 16, 16)] * wv)
                    return 0

                lax.fori_loop(0, EDGE_BATCH // 2, _srow, 0)

                pltpu.async_copy(
                    sbuf[X], acc_sh.at[dst_v.at[b]], ssem[X], add=True)

                @pl.when(g < N_BATCH // 2 - 1)
                def _():
                    _issue_gather(b + 2, X)
            return 0

        lax.fori_loop(0, N_BATCH // 2, _batch, 0)
        for X in range(2):
            pltpu.make_async_copy(
                sbuf[X], acc_sh.at[dst_v.at[0]], ssem[X]).wait()
        plsc.subcore_barrier()

        def _echunk(ro, sz):
            pltpu.sync_copy(acc_sh.at[pl.ds(ro, sz)], obuf.at[pl.ds(0, sz)])

            def _erow(r, _):
                rv = 1.0 / (obuf[r, pl.ds(DQ, 16)] + 1e-16)
                for k in range(DQ // 16):
                    gbuf0[r, pl.ds(k * 16, 16)] = (
                        obuf[r, pl.ds(k * 16, 16)] * rv
                        + bias_v[pl.ds(k * 16, 16)])
                return 0

            lax.fori_loop(0, sz, _erow, 0)

            @pl.when(c == 0)
            def _():
                if q == 0:
                    pltpu.sync_copy(gbuf0.at[pl.ds(0, sz)], oq0.at[pl.ds(ro, sz)])
                else:
                    pltpu.sync_copy(gbuf0.at[pl.ds(0, sz)], oq1.at[pl.ds(ro, sz)])

            @pl.when(c == 1)
            def _():
                if q == 0:
                    pltpu.sync_copy(gbuf0.at[pl.ds(0, sz)], oq2.at[pl.ds(ro, sz)])
                else:
                    pltpu.sync_copy(gbuf0.at[pl.ds(0, sz)], oq3.at[pl.ds(ro, sz)])

        def _et(t, _):
            _echunk(zbase + t * EPI, EPI)
            return 0

        lax.fori_loop(0, ZROWS // EPI, _et, 0)
        _echunk(zbase + ZROWS - ntail, ntail)

        if q == 0:
            plsc.subcore_barrier()


def _edge_sc(hq, asrc, adst, srcb, dstb, gmax16, bias):
    mesh = plsc.VectorSubcoreMesh(core_axis_name="c", subcore_axis_name="s")
    f32 = jnp.float32
    kern = functools.partial(
        pl.kernel,
        mesh=mesh,
        compiler_params=pltpu.CompilerParams(
            needs_layout_passes=False, use_tc_tiling_on_sc=False),
        out_type=[jax.ShapeDtypeStruct((NPAD, DQ), f32)] * 4,
        scratch_types=[
            pltpu.VMEM((N_BATCH, EDGE_BATCH), jnp.int32),
            pltpu.VMEM((N_BATCH, EDGE_BATCH), jnp.int32),
            pltpu.VMEM((EDGE_BATCH,), f32),
            pltpu.VMEM((EDGE_BATCH,), f32),
            pltpu.VMEM((N_NODES,), f32),
            pltpu.VMEM((N_NODES,), f32),
            pltpu.VMEM((EDGE_BATCH, DQ), f32),
            pltpu.VMEM((EDGE_BATCH, DQ), f32),
            pltpu.VMEM((EDGE_BATCH, DW), f32),
            pltpu.VMEM((EDGE_BATCH, DW), f32),
            pltpu.VMEM((16,), f32),
            pltpu.VMEM((DQ,), f32),
            pltpu.VMEM((EPI, DW), f32),
            pltpu.VMEM_SHARED((NPAD, DW), f32),
            pltpu.SemaphoreType.DMA,
            pltpu.SemaphoreType.DMA,
            pltpu.SemaphoreType.DMA,
            pltpu.SemaphoreType.DMA,
        ],
    )(_edge_body)
    return kern(*hq, asrc, adst, srcb, dstb, gmax16, bias)


def _gmax16(asrc, adst):
    m = jnp.max(asrc) + jnp.max(adst)
    m = jnp.where(m >= 0.0, m, m * 0.2)
    return jnp.full((16,), m, jnp.float32)


def kernel(x, edge_index, W1, att_src1, att_dst1, b1, W2, att_src2, att_dst2, b2):
    loop = jnp.arange(N_NODES, dtype=edge_index.dtype)
    padi = jnp.zeros((E_PAD - E_TOT,), edge_index.dtype)
    srcb = jnp.concatenate([edge_index[0], loop, padi]).reshape(
        N_TILES, N_BATCH, EDGE_BATCH)
    dstb = jnp.concatenate([edge_index[1], loop, padi]).reshape(
        N_TILES, N_BATCH, EDGE_BATCH)

    att2_1 = jnp.zeros((D, 8), jnp.float32).at[:, 0].set(att_src1).at[:, 1].set(att_dst1)
    att2_2 = jnp.zeros((D, 8), jnp.float32).at[:, 0].set(att_src2).at[:, 1].set(att_dst2)

    *hq, a2 = _mm1(x, W1, att2_1)
    asrc, adst = a2[:, 0], a2[:, 1]
    o1 = _edge_sc(hq, asrc, adst, srcb, dstb, _gmax16(asrc, adst), b1)

    *h2q, a2b = _mm2(o1, W2, att2_2)
    asrc2, adst2 = a2b[:N_NODES, 0], a2b[:N_NODES, 1]
    o2 = _edge_sc(h2q, asrc2, adst2, srcb, dstb, _gmax16(asrc2, adst2), b2)
    return jnp.concatenate([q[:N_NODES] for q in o2], axis=1)


# speedup vs baseline: 2.0026x; 1.8392x over previous
import functools

import jax
import jax.numpy as jnp
from jax import lax
from jax.experimental import pallas as pl
from jax.experimental.pallas import tpu as pltpu
from jax.experimental.pallas import tpu_sc as plsc

N_NODES = 10000
D = 256
DQ = 64
BN = 1000
N_TILES = 16
E_TOT = 160000 + N_NODES
EDGE_BATCH = 112
N_BATCH = 96
CHUNK = N_BATCH * EDGE_BATCH
E_PAD = N_TILES * CHUNK
NPAD = 10112
ZROWS = NPAD // N_TILES
EPI = 32
BN2 = 632


def _mm1_body(x_ref, w_ref, att2_ref, q0, q1, q2, q3, a2_ref):
    h = jnp.dot(x_ref[...], w_ref[...], preferred_element_type=jnp.float32)
    q0[...] = h[:, 0 * DQ:1 * DQ]
    q1[...] = h[:, 1 * DQ:2 * DQ]
    q2[...] = h[:, 2 * DQ:3 * DQ]
    q3[...] = h[:, 3 * DQ:4 * DQ]
    a2_ref[...] = jnp.dot(h, att2_ref[...], preferred_element_type=jnp.float32)


def _mm1(x, W, att2):
    return pl.pallas_call(
        _mm1_body,
        grid=(N_NODES // BN,),
        in_specs=[
            pl.BlockSpec((BN, D), lambda i: (i, 0)),
            pl.BlockSpec((D, D), lambda i: (0, 0)),
            pl.BlockSpec((D, 8), lambda i: (0, 0)),
        ],
        out_specs=[pl.BlockSpec((BN, DQ), lambda i: (i, 0))] * 4
        + [pl.BlockSpec((BN, 8), lambda i: (i, 0))],
        out_shape=[jax.ShapeDtypeStruct((NPAD, DQ), jnp.float32)] * 4
        + [jax.ShapeDtypeStruct((N_NODES, 8), jnp.float32)],
    )(x, W, att2)


def _mm2_body(x0, x1, x2, x3, w_ref, att2_ref, q0, q1, q2, q3, a2_ref):
    h = jnp.dot(jnp.maximum(x0[...], 0.0), w_ref[0 * DQ:1 * DQ, :],
                preferred_element_type=jnp.float32)
    h += jnp.dot(jnp.maximum(x1[...], 0.0), w_ref[1 * DQ:2 * DQ, :],
                 preferred_element_type=jnp.float32)
    h += jnp.dot(jnp.maximum(x2[...], 0.0), w_ref[2 * DQ:3 * DQ, :],
                 preferred_element_type=jnp.float32)
    h += jnp.dot(jnp.maximum(x3[...], 0.0), w_ref[3 * DQ:4 * DQ, :],
                 preferred_element_type=jnp.float32)
    q0[...] = h[:, 0 * DQ:1 * DQ]
    q1[...] = h[:, 1 * DQ:2 * DQ]
    q2[...] = h[:, 2 * DQ:3 * DQ]
    q3[...] = h[:, 3 * DQ:4 * DQ]
    a2_ref[...] = jnp.dot(h, att2_ref[...], preferred_element_type=jnp.float32)


def _mm2(xq, W, att2):
    return pl.pallas_call(
        _mm2_body,
        grid=(NPAD // BN2,),
        in_specs=[pl.BlockSpec((BN2, DQ), lambda i: (i, 0))] * 4
        + [
            pl.BlockSpec((D, D), lambda i: (0, 0)),
            pl.BlockSpec((D, 8), lambda i: (0, 0)),
        ],
        out_specs=[pl.BlockSpec((BN2, DQ), lambda i: (i, 0))] * 4
        + [pl.BlockSpec((BN2, 8), lambda i: (i, 0))],
        out_shape=[jax.ShapeDtypeStruct((NPAD, DQ), jnp.float32)] * 4
        + [jax.ShapeDtypeStruct((NPAD, 8), jnp.float32)],
    )(*xq, W, att2)


DW = DQ + 16


def _edge_body(hq0, hq1, hq2, hq3, asrc_h, adst_h, srcb_h, dstb_h,
               gmax_h, bias_h, oq0, oq1, oq2, oq3,
               src_v, dst_v, wb0, wb1, asrc_v, adst_v, gbuf0, gbuf1,
               sbuf0, sbuf1, gmax_v, bias_v, obuf, acc_sh,
               gsem0, gsem1, ssem0, ssem1):
    c = lax.axis_index("c")
    s = lax.axis_index("s")
    gbuf = (gbuf0, gbuf1)
    sbuf = (sbuf0, sbuf1)
    gsem = (gsem0, gsem1)
    wb = (wb0, wb1)
    ssem = (ssem0, ssem1)

    pltpu.sync_copy(srcb_h.at[s], src_v)
    pltpu.sync_copy(dstb_h.at[s], dst_v)
    pltpu.sync_copy(asrc_h, asrc_v)
    pltpu.sync_copy(adst_h, adst_v)
    pltpu.sync_copy(gmax_h, gmax_v)

    zero16 = jnp.zeros((16,), jnp.float32)
    lanes = lax.iota(jnp.int32, 16)
    zbase = s * ZROWS
    gv = gmax_v[...]
    ntail = ZROWS - (ZROWS // EPI) * EPI

    def _zrow(r, _):
        for k in range(DW // 16):
            obuf[r, pl.ds(k * 16, 16)] = zero16
        return 0

    for q in range(2):

        def _issue_gather(bidx, X):
            @pl.when(c == 0)
            def _():
                pltpu.async_copy(
                    (hq0 if q == 0 else hq1).at[src_v.at[bidx]],
                    gbuf[X], gsem[X])

            @pl.when(c == 1)
            def _():
                pltpu.async_copy(
                    (hq2 if q == 0 else hq3).at[src_v.at[bidx]],
                    gbuf[X], gsem[X])

        _issue_gather(0, 0)
        _issue_gather(1, 1)

        @pl.when(c == 0)
        def _():
            pltpu.sync_copy(bias_h.at[pl.ds(q * DQ, DQ)], bias_v)

        @pl.when(c == 1)
        def _():
            pltpu.sync_copy(bias_h.at[pl.ds((2 + q) * DQ, DQ)], bias_v)

        lax.fori_loop(0, EPI, _zrow, 0)

        def _zcopy(t, _):
            pltpu.sync_copy(obuf, acc_sh.at[pl.ds(zbase + t * EPI, EPI)])
            return 0

        lax.fori_loop(0, ZROWS // EPI, _zcopy, 0)
        pltpu.sync_copy(obuf.at[pl.ds(0, ntail)],
                        acc_sh.at[pl.ds(zbase + ZROWS - ntail, ntail)])
        plsc.subcore_barrier()

        def _batch(g, _):
            for X in range(2):
                b = 2 * g + X
                pltpu.make_async_copy(
                    hq0.at[src_v.at[0]], gbuf[X], gsem[X]).wait()
                @pl.when(g > 0)
                def _():
                    pltpu.make_async_copy(
                        sbuf[X], acc_sh.at[dst_v.at[0]], ssem[X]).wait()

                eid0 = (s * CHUNK + b * EDGE_BATCH)
                for k in range(EDGE_BATCH // 16):
                    sv = src_v[b, pl.ds(k * 16, 16)]
                    dv = dst_v[b, pl.ds(k * 16, 16)]
                    al = (plsc.load_gather(asrc_v, [sv])
                          + plsc.load_gather(adst_v, [dv]))
                    al = jnp.where(al >= 0.0, al, al * 0.2)
                    wv = jnp.exp(al - gv)
                    eid = jnp.full((16,), eid0 + k * 16, jnp.int32) + lanes
                    wv = jnp.where(eid < E_TOT, wv, 0.0)
                    wb[X][pl.ds(k * 16, 16)] = wv

                @functools.partial(plsc.parallel_loop, 0, EDGE_BATCH, unroll=4)
                def _srow(r):
                    wv = plsc.load_gather(
                        wb[X], [jnp.full((16,), r, jnp.int32)])
                    sbuf[X][r, pl.ds(DQ, 16)] = wv
                    for k in range(DQ // 16):
                        sbuf[X][r, pl.ds(k * 16, 16)] = (
                            gbuf[X][r, pl.ds(k * 16, 16)] * wv)

                pltpu.async_copy(
                    sbuf[X], acc_sh.at[dst_v.at[b]], ssem[X], add=True)

                @pl.when(g < N_BATCH // 2 - 1)
                def _():
                    _issue_gather(b + 2, X)
            return 0

        lax.fori_loop(0, N_BATCH // 2, _batch, 0)
        for X in range(2):
            pltpu.make_async_copy(
                sbuf[X], acc_sh.at[dst_v.at[0]], ssem[X]).wait()
        plsc.subcore_barrier()

        def _echunk(ro, sz):
            pltpu.sync_copy(acc_sh.at[pl.ds(ro, sz)], obuf.at[pl.ds(0, sz)])

            def _erow(r, _):
                rv = 1.0 / (obuf[r, pl.ds(DQ, 16)] + 1e-16)
                for k in range(DQ // 16):
                    gbuf0[r, pl.ds(k * 16, 16)] = (
                        obuf[r, pl.ds(k * 16, 16)] * rv
                        + bias_v[pl.ds(k * 16, 16)])
                return 0

            lax.fori_loop(0, sz, _erow, 0)

            @pl.when(c == 0)
            def _():
                if q == 0:
                    pltpu.sync_copy(gbuf0.at[pl.ds(0, sz)], oq0.at[pl.ds(ro, sz)])
                else:
                    pltpu.sync_copy(gbuf0.at[pl.ds(0, sz)], oq1.at[pl.ds(ro, sz)])

            @pl.when(c == 1)
            def _():
                if q == 0:
                    pltpu.sync_copy(gbuf0.at[pl.ds(0, sz)], oq2.at[pl.ds(ro, sz)])
                else:
                    pltpu.sync_copy(gbuf0.at[pl.ds(0, sz)], oq3.at[pl.ds(ro, sz)])

        def _et(t, _):
            _echunk(zbase + t * EPI, EPI)
            return 0

        lax.fori_loop(0, ZROWS // EPI, _et, 0)
        _echunk(zbase + ZROWS - ntail, ntail)

        if q == 0:
            plsc.subcore_barrier()


def _edge_sc(hq, asrc, adst, srcb, dstb, gmax16, bias):
    mesh = plsc.VectorSubcoreMesh(core_axis_name="c", subcore_axis_name="s")
    f32 = jnp.float32
    kern = functools.partial(
        pl.kernel,
        mesh=mesh,
        compiler_params=pltpu.CompilerParams(
            needs_layout_passes=False, use_tc_tiling_on_sc=False),
        out_type=[jax.ShapeDtypeStruct((NPAD, DQ), f32)] * 4,
        scratch_types=[
            pltpu.VMEM((N_BATCH, EDGE_BATCH), jnp.int32),
            pltpu.VMEM((N_BATCH, EDGE_BATCH), jnp.int32),
            pltpu.VMEM((EDGE_BATCH,), f32),
            pltpu.VMEM((EDGE_BATCH,), f32),
            pltpu.VMEM((N_NODES,), f32),
            pltpu.VMEM((N_NODES,), f32),
            pltpu.VMEM((EDGE_BATCH, DQ), f32),
            pltpu.VMEM((EDGE_BATCH, DQ), f32),
            pltpu.VMEM((EDGE_BATCH, DW), f32),
            pltpu.VMEM((EDGE_BATCH, DW), f32),
            pltpu.VMEM((16,), f32),
            pltpu.VMEM((DQ,), f32),
            pltpu.VMEM((EPI, DW), f32),
            pltpu.VMEM_SHARED((NPAD, DW), f32),
            pltpu.SemaphoreType.DMA,
            pltpu.SemaphoreType.DMA,
            pltpu.SemaphoreType.DMA,
            pltpu.SemaphoreType.DMA,
        ],
    )(_edge_body)
    return kern(*hq, asrc, adst, srcb, dstb, gmax16, bias)


def _gmax16(asrc, adst):
    m = jnp.max(asrc) + jnp.max(adst)
    m = jnp.where(m >= 0.0, m, m * 0.2)
    return jnp.full((16,), m, jnp.float32)


def kernel(x, edge_index, W1, att_src1, att_dst1, b1, W2, att_src2, att_dst2, b2):
    loop = jnp.arange(N_NODES, dtype=edge_index.dtype)
    padi = jnp.zeros((E_PAD - E_TOT,), edge_index.dtype)
    srcb = jnp.concatenate([edge_index[0], loop, padi]).reshape(
        N_TILES, N_BATCH, EDGE_BATCH)
    dstb = jnp.concatenate([edge_index[1], loop, padi]).reshape(
        N_TILES, N_BATCH, EDGE_BATCH)

    att2_1 = jnp.zeros((D, 8), jnp.float32).at[:, 0].set(att_src1).at[:, 1].set(att_dst1)
    att2_2 = jnp.zeros((D, 8), jnp.float32).at[:, 0].set(att_src2).at[:, 1].set(att_dst2)

    *hq, a2 = _mm1(x, W1, att2_1)
    asrc, adst = a2[:, 0], a2[:, 1]
    o1 = _edge_sc(hq, asrc, adst, srcb, dstb, _gmax16(asrc, adst), b1)

    *h2q, a2b = _mm2(o1, W2, att2_2)
    asrc2, adst2 = a2b[:N_NODES, 0], a2b[:N_NODES, 1]
    o2 = _edge_sc(h2q, asrc2, adst2, srcb, dstb, _gmax16(asrc2, adst2), b2)
    return jnp.concatenate([q[:N_NODES] for q in o2], axis=1)
